# baseline (speedup 1.0000x reference)
import functools

import jax
import jax.numpy as jnp
from jax import lax
from jax.experimental import pallas as pl
from jax.experimental.pallas import tpu as pltpu
from jax.experimental.pallas import tpu_sc as plsc

V = 100000
D = 64
R = 8

NC = 2
NS = 16
L = 16

RED_STEP = 6240
RED_FETCH = 6400
NBLK_STD, NBLK_LAST = 390, 400

CH = 80
NCHUNK = V // CH


def _body(x_hbm, w_hbm, a_hbm, b_hbm, out_hbm,
          xr_v, av, bv, st_v, red_v, idx_v, rows_v, shared, sem):
    cid = lax.axis_index("c")
    sid = lax.axis_index("s")
    wid = sid * NC + cid

    iota = lax.iota(jnp.int32, L)
    half = iota < 8

    red_lo = pl.multiple_of(sid * RED_STEP, 16)
    pltpu.sync_copy(x_hbm.at[pl.ds(red_lo, RED_FETCH)], xr_v)
    pltpu.sync_copy(a_hbm.at[pl.ds(red_lo * R, RED_FETCH * R)], av)
    pltpu.sync_copy(b_hbm, bv)

    pair_idx = [jnp.where(half, 2 * j, 2 * j + 1) for j in range(8)]

    def blk(t, accs):
        off = pl.multiple_of(t * L, 16)
        xf = xr_v[pl.ds(off, L)].astype(jnp.float32)
        out = []
        for j in range(8):
            xg = xf.at[pair_idx[j]].get(mode="promise_in_bounds")
            a = av[pl.ds(pl.multiple_of(t * 128 + j * L, 16), L)]
            out.append(accs[j] + xg * a)
        return tuple(out)

    nblk = jnp.where(sid == NS - 1, NBLK_LAST, NBLK_STD)
    zero = jnp.zeros((L,), jnp.float32)
    accs = lax.fori_loop(0, nblk, blk, (zero,) * 8)
    acc = ((accs[0] + accs[1]) + (accs[2] + accs[3])) + \
          ((accs[4] + accs[5]) + (accs[6] + accs[7]))

    st_v[...] = acc
    pltpu.sync_copy(st_v, shared.at[sid])
    plsc.subcore_barrier()
    pltpu.sync_copy(shared, red_v)
    tot = red_v[0, :]
    for i in range(1, NS):
        tot = tot + red_v[i, :]
    swap = jnp.where(half, iota + 8, iota - 8)
    s = tot + tot.at[swap].get(mode="promise_in_bounds")

    lora = []
    for k in range(4):
        lk = zero
        for r in range(R):
            sr = s.at[jnp.full((L,), r, jnp.int32)].get(
                mode="promise_in_bounds")
            lk = lk + sr * bv[pl.ds(r * D + k * L, L)]
        lora.append(lk)

    nch = (NCHUNK - wid + (NC * NS - 1)) // (NC * NS)

    def chunk(t, carry):
        base = pl.multiple_of((wid + t * NC * NS) * CH, 16)
        pltpu.sync_copy(x_hbm.at[pl.ds(base, CH)], idx_v)
        pltpu.async_copy(w_hbm.at[idx_v], rows_v, sem).wait()

        def rows4(rr, c2):
            r0 = pl.multiple_of(rr * 4, 4)
            for u in range(4):
                for k in range(4):
                    rows_v[r0 + u, pl.ds(k * L, L)] = (
                        rows_v[r0 + u, pl.ds(k * L, L)] + lora[k])
            return c2

        lax.fori_loop(0, CH // 4, rows4, 0)
        pltpu.sync_copy(rows_v, out_hbm.at[pl.ds(base, CH), :])
        return carry

    lax.fori_loop(0, nch, chunk, 0)


@functools.partial(jax.jit, static_argnums=())
def _emb_lora(x, weight, a_flat, b_flat):
    mesh = plsc.VectorSubcoreMesh(core_axis_name="c", subcore_axis_name="s")
    k = pl.kernel(
        _body,
        out_type=jax.ShapeDtypeStruct((V, D), jnp.float32),
        mesh=mesh,
        scratch_types=[
            pltpu.VMEM((RED_FETCH,), jnp.int32),
            pltpu.VMEM((RED_FETCH * R,), jnp.float32),
            pltpu.VMEM((R * D,), jnp.float32),
            pltpu.VMEM((L,), jnp.float32),
            pltpu.VMEM((NS, L), jnp.float32),
            pltpu.VMEM((CH,), jnp.int32),
            pltpu.VMEM((CH, D), jnp.float32),
            pltpu.VMEM_SHARED((NS, L), jnp.float32),
            pltpu.SemaphoreType.DMA,
        ],
        compiler_params=pltpu.CompilerParams(use_tc_tiling_on_sc=False),
    )
    return k(x, weight, a_flat, b_flat)


def kernel(x, weight, A, B):
    return _emb_lora(x, weight, A.reshape(-1), B.reshape(-1))


# speedup vs baseline: 1.1786x; 1.1786x over previous
import functools

import jax
import jax.numpy as jnp
from jax import lax
from jax.experimental import pallas as pl
from jax.experimental.pallas import tpu as pltpu
from jax.experimental.pallas import tpu_sc as plsc

V = 100000
D = 64
R = 8

NC = 2
NS = 16
L = 16

RED_STEP = 6240
RED_FETCH = 6400
NBLK_STD, NBLK_LAST = 390, 400

CH = 80
STRIDE = 3120
NCH = 41
IB = NCH * CH


def _body(x_hbm, w_hbm, a_hbm, b_hbm, out_hbm,
          xr_v, av, bv, st_v, red_v, idxs_v, g0, g1, s0, s1, shared,
          sg0, sg1, ss0, ss1):
    cid = lax.axis_index("c")
    sid = lax.axis_index("s")
    wid = sid * NC + cid

    iota = lax.iota(jnp.int32, L)
    half = iota < 8

    gbuf, sbuf = (g0, g1), (s0, s1)
    gsem, ssem = (sg0, sg1), (ss0, ss1)
    base = pl.multiple_of(wid * STRIDE, 16)

    def idx_slice(c):
        return idxs_v.at[pl.ds(pl.multiple_of(c * CH, 16), CH)]

    def gstart(c, p):
        pltpu.async_copy(w_hbm.at[idx_slice(c)], gbuf[p], gsem[p])

    def gwait(c, p):
        pltpu.make_async_copy(w_hbm.at[idx_slice(c)], gbuf[p], gsem[p]).wait()

    def out_slice(c):
        return out_hbm.at[pl.ds(pl.multiple_of(base + c * CH, 16), CH), :]

    def sstart(c, p):
        pltpu.async_copy(sbuf[p], out_slice(c), ssem[p])

    def swait(c, p):
        pltpu.make_async_copy(sbuf[p], out_slice(c), ssem[p]).wait()

    pltpu.sync_copy(x_hbm.at[pl.ds(base, IB)], idxs_v)
    gstart(0, 0)
    gstart(1, 1)

    red_lo = pl.multiple_of(sid * RED_STEP, 16)
    pltpu.sync_copy(x_hbm.at[pl.ds(red_lo, RED_FETCH)], xr_v)
    pltpu.sync_copy(a_hbm.at[pl.ds(red_lo * R, RED_FETCH * R)], av)
    pltpu.sync_copy(b_hbm, bv)

    pair_idx = [jnp.where(half, 2 * j, 2 * j + 1) for j in range(8)]

    def blk(t, accs):
        off = pl.multiple_of(t * L, 16)
        xf = xr_v[pl.ds(off, L)].astype(jnp.float32)
        out = []
        for j in range(8):
            xg = xf.at[pair_idx[j]].get(mode="promise_in_bounds")
            a = av[pl.ds(pl.multiple_of(t * 128 + j * L, 16), L)]
            out.append(accs[j] + xg * a)
        return tuple(out)

    nblk = jnp.where(sid == NS - 1, NBLK_LAST, NBLK_STD)
    zero = jnp.zeros((L,), jnp.float32)
    accs = lax.fori_loop(0, nblk, blk, (zero,) * 8)
    acc = ((accs[0] + accs[1]) + (accs[2] + accs[3])) + \
          ((accs[4] + accs[5]) + (accs[6] + accs[7]))

    st_v[...] = acc
    pltpu.sync_copy(st_v, shared.at[sid])
    plsc.subcore_barrier()
    pltpu.sync_copy(shared, red_v)
    tot = red_v[0, :]
    for i in range(1, NS):
        tot = tot + red_v[i, :]
    swap = jnp.where(half, iota + 8, iota - 8)
    s = tot + tot.at[swap].get(mode="promise_in_bounds")

    lora = []
    for k in range(4):
        lk = zero
        for r in range(R):
            sr = s.at[jnp.full((L,), r, jnp.int32)].get(
                mode="promise_in_bounds")
            lk = lk + sr * bv[pl.ds(r * D + k * L, L)]
        lora.append(lk)

    def add_rows(p):
        def rows4(rr, c2):
            r0 = pl.multiple_of(rr * 4, 4)
            for u in range(4):
                for k in range(4):
                    sbuf[p][r0 + u, pl.ds(k * L, L)] = (
                        gbuf[p][r0 + u, pl.ds(k * L, L)] + lora[k])
            return c2
        lax.fori_loop(0, CH // 4, rows4, 0)

    def stage(c, p, do_swait, do_prefetch):
        gwait(c, p)
        if do_swait == "always":
            swait(c - 2, p)
        elif do_swait is not None:

            @pl.when(do_swait)
            def _():
                swait(c - 2, p)

        add_rows(p)
        if do_prefetch:
            gstart(c + 2, p)
        sstart(c, p)

    def pair(m, carry):
        c0 = m * 2
        stage(c0, 0, m > 0, True)
        stage(c0 + 1, 1, m > 0, True)
        return carry

    lax.fori_loop(0, 19, pair, 0)
    stage(38, 0, "always", True)
    stage(39, 1, "always", False)
    stage(40, 0, "always", False)
    swait(39, 1)
    swait(40, 0)


@functools.partial(jax.jit, static_argnums=())
def _emb_lora(x, weight, a_flat, b_flat):
    mesh = plsc.VectorSubcoreMesh(core_axis_name="c", subcore_axis_name="s")
    k = pl.kernel(
        _body,
        out_type=jax.ShapeDtypeStruct((V, D), jnp.float32),
        mesh=mesh,
        scratch_types=[
            pltpu.VMEM((RED_FETCH,), jnp.int32),
            pltpu.VMEM((RED_FETCH * R,), jnp.float32),
            pltpu.VMEM((R * D,), jnp.float32),
            pltpu.VMEM((L,), jnp.float32),
            pltpu.VMEM((NS, L), jnp.float32),
            pltpu.VMEM((IB,), jnp.int32),
            pltpu.VMEM((CH, D), jnp.float32),
            pltpu.VMEM((CH, D), jnp.float32),
            pltpu.VMEM((CH, D), jnp.float32),
            pltpu.VMEM((CH, D), jnp.float32),
            pltpu.VMEM_SHARED((NS, L), jnp.float32),
            pltpu.SemaphoreType.DMA,
            pltpu.SemaphoreType.DMA,
            pltpu.SemaphoreType.DMA,
            pltpu.SemaphoreType.DMA,
        ],
        compiler_params=pltpu.CompilerParams(use_tc_tiling_on_sc=False),
    )
    return k(x, weight, a_flat, b_flat)


def kernel(x, weight, A, B):
    return _emb_lora(x, weight, A.reshape(-1), B.reshape(-1))
